# baseline (speedup 1.0000x reference)
import functools

import jax
import jax.numpy as jnp
from jax import lax
from jax.experimental import pallas as pl
from jax.experimental.pallas import tpu as pltpu
from jax.experimental.pallas import tpu_sc as plsc

NC, NS = 2, 16
NW = NC * NS


def _sc_gather(table, idx, B, D):
    b_per_w = B // NW
    mesh = plsc.VectorSubcoreMesh(
        core_axis_name="c", subcore_axis_name="s",
        num_cores=NC, num_subcores=NS)

    @functools.partial(
        pl.kernel, mesh=mesh,
        compiler_params=pltpu.CompilerParams(use_tc_tiling_on_sc=False),
        out_type=jax.ShapeDtypeStruct((B, D), jnp.float32),
        scratch_types=[
            pltpu.VMEM((b_per_w,), jnp.int32),
            pltpu.VMEM((b_per_w, D), jnp.float32),
            pltpu.SemaphoreType.DMA,
        ],
    )
    def k(table_hbm, idx_hbm, out_hbm, idx_v, rows_v, sem):
        wid = lax.axis_index("s") * NC + lax.axis_index("c")
        base = wid * b_per_w
        pltpu.sync_copy(idx_hbm.at[pl.ds(base, b_per_w)], idx_v)
        pltpu.async_copy(table_hbm.at[idx_v], rows_v, sem).wait()
        pltpu.sync_copy(rows_v, out_hbm.at[pl.ds(base, b_per_w)])

    return k(table, idx)


def _tc_body(emb_ref, s0_ref, s1_ref, s2_ref, w_ref, b_ref, out_ref):
    w = w_ref[...]
    part = (emb_ref[...] * w[:, 0:64]
            + s0_ref[...] * w[:, 64:128]
            + s1_ref[...] * w[:, 128:192]
            + s2_ref[...] * w[:, 192:256])
    acc = jnp.sum(part, axis=1) + b_ref[0, 0]
    out_ref[...] = jax.nn.sigmoid(acc)[:, None]


def kernel(node_idx, signal_0, signal_1, signal_2, node_embed, fc_w, fc_b):
    B, D = signal_0.shape
    emb_rows = _sc_gather(node_embed, node_idx.astype(jnp.int32), B, D)

    BLK = 2048
    grid = (B // BLK,)
    sig_spec = pl.BlockSpec((BLK, D), lambda i: (i, 0))
    p = pl.pallas_call(
        _tc_body,
        grid=grid,
        in_specs=[sig_spec, sig_spec, sig_spec, sig_spec,
                  pl.BlockSpec((1, 4 * D), lambda i: (0, 0)),
                  pl.BlockSpec((1, 1), lambda i: (0, 0))],
        out_specs=pl.BlockSpec((BLK, 1), lambda i: (i, 0)),
        out_shape=jax.ShapeDtypeStruct((B, 1), jnp.float32),
    )(emb_rows, signal_0, signal_1, signal_2, fc_w, fc_b.reshape(1, 1))

    return (p, jnp.float32(0.0))
